# baseline (speedup 1.0000x reference)
import jax
import jax.numpy as jnp
from jax.experimental import pallas as pl
from jax.sharding import Mesh, PartitionSpec as P

_BM = 512


def _sage_block(adj_ref, x_ref, xs_ref, wrel_ref, wroot_ref, b_ref, out_ref):
    a = adj_ref[...].astype(jnp.bfloat16)
    deg = jnp.clip(jnp.sum(a, axis=1, keepdims=True, dtype=jnp.float32),
                   1.0, None)
    agg = jnp.dot(a, x_ref[...].astype(jnp.bfloat16),
                  preferred_element_type=jnp.float32)
    agg = agg / deg
    out = (jnp.dot(agg, wrel_ref[...], preferred_element_type=jnp.float32)
           + jnp.dot(xs_ref[...], wroot_ref[...],
                     preferred_element_type=jnp.float32)
           + b_ref[...])
    nrm = jnp.sqrt(jnp.sum(out * out, axis=1, keepdims=True))
    out = out / jnp.clip(nrm, 1e-12, None)
    out_ref[...] = jnp.where(out >= 0, out, 0.01 * out)


def _sage_shard(adj_s, x_full, x_s, W_rel, W_root, b2):
    rows, n = adj_s.shape
    c_in, c_out = W_rel.shape
    return pl.pallas_call(
        _sage_block,
        grid=(rows // _BM,),
        in_specs=[
            pl.BlockSpec((_BM, n), lambda i: (i, 0)),
            pl.BlockSpec((n, c_in), lambda i: (0, 0)),
            pl.BlockSpec((_BM, c_in), lambda i: (i, 0)),
            pl.BlockSpec((c_in, c_out), lambda i: (0, 0)),
            pl.BlockSpec((c_in, c_out), lambda i: (0, 0)),
            pl.BlockSpec((1, c_out), lambda i: (0, 0)),
        ],
        out_specs=pl.BlockSpec((_BM, c_out), lambda i: (i, 0)),
        out_shape=jax.ShapeDtypeStruct((rows, c_out), jnp.float32),
    )(adj_s, x_full, x_s, W_rel, W_root, b2)


def kernel(x, adj, W_rel, W_root, b):
    B, N, C_in = x.shape
    C_out = W_rel.shape[1]
    x2 = x.reshape(N, C_in)
    adj2 = adj.reshape(N, N)
    b2 = b.reshape(1, C_out)

    devs = jax.devices()
    n_shard = 2 if (len(devs) >= 2 and N % (2 * _BM) == 0) else 1
    mesh = Mesh(devs[:n_shard], ("i",))
    sharded = jax.shard_map(
        _sage_shard,
        mesh=mesh,
        in_specs=(P("i", None), P(None, None), P("i", None),
                  P(None, None), P(None, None), P(None, None)),
        out_specs=P("i", None),
        check_vma=False,
    )
    out = jax.jit(sharded)(adj2, x2, x2, W_rel, W_root, b2)
    return out.reshape(B, N, C_out)


# speedup vs baseline: 18.8239x; 18.8239x over previous
import jax
import jax.numpy as jnp
from jax.experimental import pallas as pl

_BM = 512


def _sage_block(adj_ref, x_ref, wrel_ref, wroot_ref, b_ref, out_ref):
    i = pl.program_id(0)
    a = adj_ref[...].astype(jnp.bfloat16)
    deg = jnp.clip(jnp.sum(a, axis=1, keepdims=True, dtype=jnp.float32),
                   1.0, None)
    agg = jnp.dot(a, x_ref[...].astype(jnp.bfloat16),
                  preferred_element_type=jnp.float32)
    agg = agg / deg
    x_blk = x_ref[pl.ds(i * _BM, _BM), :]
    out = (jnp.dot(agg, wrel_ref[...], preferred_element_type=jnp.float32)
           + jnp.dot(x_blk, wroot_ref[...], preferred_element_type=jnp.float32)
           + b_ref[...])
    nrm = jnp.sqrt(jnp.sum(out * out, axis=1, keepdims=True))
    out = out / jnp.clip(nrm, 1e-12, None)
    out_ref[...] = jnp.where(out >= 0, out, 0.01 * out)


def kernel(x, adj, W_rel, W_root, b):
    B, N, C_in = x.shape
    C_out = W_rel.shape[1]
    x2 = x.reshape(N, C_in)
    adj2 = adj.reshape(N, N)
    b2 = b.reshape(1, C_out)
    out = pl.pallas_call(
        _sage_block,
        grid=(N // _BM,),
        in_specs=[
            pl.BlockSpec((_BM, N), lambda i: (i, 0)),
            pl.BlockSpec((N, C_in), lambda i: (0, 0)),
            pl.BlockSpec((C_in, C_out), lambda i: (0, 0)),
            pl.BlockSpec((C_in, C_out), lambda i: (0, 0)),
            pl.BlockSpec((1, C_out), lambda i: (0, 0)),
        ],
        out_specs=pl.BlockSpec((_BM, C_out), lambda i: (i, 0)),
        out_shape=jax.ShapeDtypeStruct((N, C_out), jnp.float32),
    )(adj2, x2, W_rel, W_root, b2)
    return out.reshape(B, N, C_out)
